# Initial kernel scaffold; baseline (speedup 1.0000x reference)
import jax
import jax.numpy as jnp
from jax.experimental import pallas as pl


def kernel(h, k_weight, q_weight, v_weight, a_weight, relation_pri, relation_att, relation_msg, skip, edge_index, node_type, edge_type):
    raise NotImplementedError("write your pallas kernel here")


# speedup vs baseline: 6.7634x; 6.7634x over previous
import functools
import math

import jax
import jax.numpy as jnp
from jax import lax
from jax.experimental import pallas as pl
from jax.experimental.pallas import tpu as pltpu
from jax.experimental.pallas import tpu_sc as plsc

N = 10000
E = 160000
D = 128
NT = 4
NR = 4

NC = 2
NS = 16
NW = NC * NS
L = 16

CH = 128
NCHUNK = E // CH
CHC = 64
NCHUNKC = E // CHC
N_PAD = 10240
NSL = N_PAD // NS

_INV_SQRT_DK = 1.0 / math.sqrt(float(D))
_NEG_BIG = -3.0e38


_BN = 1000


def _proj_body(nt_ref, h_ref, kw_ref, qw_ref, vw_ref, att_ref, msg_ref,
               q_out, k_out, v_out):
    h = h_ref[...]
    nt = nt_ref[...]
    kn = jnp.zeros_like(h)
    qn = jnp.zeros_like(h)
    vn = jnp.zeros_like(h)
    for t in range(NT):
        m = (nt == t).astype(jnp.float32)
        kn += m * jnp.dot(h, kw_ref[t], preferred_element_type=jnp.float32)
        qn += m * jnp.dot(h, qw_ref[t], preferred_element_type=jnp.float32)
        vn += m * jnp.dot(h, vw_ref[t], preferred_element_type=jnp.float32)
    q_out[...] = qn
    for r in range(NR):
        k_out[r] = jnp.dot(kn, att_ref[r], preferred_element_type=jnp.float32)
        v_out[r] = jnp.dot(vn, msg_ref[r], preferred_element_type=jnp.float32)


def _proj(nt2, h, kw, qw, vw, att, msg):
    grid = N // _BN
    return pl.pallas_call(
        _proj_body,
        grid=(grid,),
        in_specs=[
            pl.BlockSpec((_BN, 1), lambda i: (i, 0)),
            pl.BlockSpec((_BN, D), lambda i: (i, 0)),
            pl.BlockSpec((NT, D, D), lambda i: (0, 0, 0)),
            pl.BlockSpec((NT, D, D), lambda i: (0, 0, 0)),
            pl.BlockSpec((NT, D, D), lambda i: (0, 0, 0)),
            pl.BlockSpec((NR, D, D), lambda i: (0, 0, 0)),
            pl.BlockSpec((NR, D, D), lambda i: (0, 0, 0)),
        ],
        out_specs=[
            pl.BlockSpec((_BN, D), lambda i: (i, 0)),
            pl.BlockSpec((NR, _BN, D), lambda i: (0, i, 0)),
            pl.BlockSpec((NR, _BN, D), lambda i: (0, i, 0)),
        ],
        out_shape=[
            jax.ShapeDtypeStruct((N, D), jnp.float32),
            jax.ShapeDtypeStruct((NR, N, D), jnp.float32),
            jax.ShapeDtypeStruct((NR, N, D), jnp.float32),
        ],
    )(nt2, h, kw, qw, vw, att, msg)


_MESH = plsc.VectorSubcoreMesh(core_axis_name="c", subcore_axis_name="s")


def _attn_body(kflat_hbm, qnode_hbm, src_hbm, dst_hbm, et_hbm, pri_hbm,
               attn_hbm, mpart_hbm,
               m_loc, kidx, dst_v, src_v, et_v, k_rows, q_rows, attn_v,
               pri_v, pbuf, kb, vb, sem1, sem2):
    c = lax.axis_index("c")
    s = lax.axis_index("s")
    w = s * NC + c
    lane = lax.iota(jnp.int32, L)
    cols = [lane + j * L for j in range(D // L)]
    tcol = lane * L
    zero = jnp.zeros((L,), jnp.float32)

    def init_body(i, _):
        m_loc[pl.ds(i * L, L)] = jnp.full((L,), _NEG_BIG, jnp.float32)
        return 0
    lax.fori_loop(0, N_PAD // L, init_body, 0)

    pltpu.sync_copy(pri_hbm, pri_v)
    kb[pl.ds(0, L)] = jnp.full((L,), -1, jnp.int32)
    kb[pl.ds(2 * L, L)] = jnp.full((L,), -2, jnp.int32)
    vb[pl.ds(0, L)] = jnp.full((L,), _NEG_BIG, jnp.float32)

    nchunks = jnp.where(w < NCHUNK - (NCHUNK // NW) * NW,
                        NCHUNK // NW + 1, NCHUNK // NW)

    def chunk_body(i, _):
        base = (w + i * NW) * CH
        pltpu.sync_copy(src_hbm.at[pl.ds(base, CH)], src_v)
        pltpu.sync_copy(dst_hbm.at[pl.ds(base, CH)], dst_v)
        pltpu.sync_copy(et_hbm.at[pl.ds(base, CH)], et_v)

        def idx_body(j, _):
            sl = pl.ds(j * L, L)
            kidx[sl] = src_v[sl] + et_v[sl] * N
            return 0
        lax.fori_loop(0, CH // L, idx_body, 0)

        cp1 = pltpu.async_copy(kflat_hbm.at[kidx], k_rows, sem1)
        cp2 = pltpu.async_copy(qnode_hbm.at[dst_v], q_rows, sem2)
        cp1.wait()
        cp2.wait()

        def group_body(g, _):
            eb = g * L
            sl_g = pl.ds(eb, L)
            for el in range(L):
                e = eb + el
                p = zero
                for j in range(D // L):
                    kk = k_rows[e, pl.ds(j * L, L)]
                    qq = q_rows[e, pl.ds(j * L, L)]
                    p = p + kk * qq
                pbuf[pl.ds(el * L, L)] = p
            acc = zero
            for l in range(L):
                acc = acc + plsc.load_gather(pbuf, [tcol + l])
            et16 = et_v[sl_g]
            pri16 = plsc.load_gather(pri_v, [et16])
            attn16 = acc * (pri16 * _INV_SQRT_DK)
            attn_v[sl_g] = attn16
            dst16 = dst_v[sl_g]
            sk = plsc.sort_key_val(dst16, attn16)
            sv = sk[1]
            sk = sk[0]
            kb[pl.ds(L, L)] = sk
            vb[pl.ds(L, L)] = sv
            for shift in (1, 2, 4, 8):
                ks = kb[pl.ds(L - shift, L)]
                vs = vb[pl.ds(L - shift, L)]
                sv = jnp.maximum(sv, jnp.where(ks == sk, vs, _NEG_BIG))
                vb[pl.ds(L, L)] = sv
            nxt = kb[pl.ds(L + 1, L)]
            is_last = sk != nxt
            cur = plsc.load_gather(m_loc, [sk])
            plsc.store_scatter(m_loc, [sk], jnp.maximum(cur, sv),
                               mask=is_last)
            return 0
        lax.fori_loop(0, CH // L, group_body, 0)

        pltpu.sync_copy(attn_v, attn_hbm.at[pl.ds(base, CH)])
        return 0
    lax.fori_loop(0, nchunks, chunk_body, 0)

    pltpu.sync_copy(m_loc, mpart_hbm.at[w])


_attn_kernel = pl.kernel(
    _attn_body,
    out_type=[
        jax.ShapeDtypeStruct((E,), jnp.float32),
        jax.ShapeDtypeStruct((NW, N_PAD), jnp.float32),
    ],
    mesh=_MESH,
    scratch_types=[
        pltpu.VMEM((N_PAD,), jnp.float32),
        pltpu.VMEM((CH,), jnp.int32),
        pltpu.VMEM((CH,), jnp.int32),
        pltpu.VMEM((CH,), jnp.int32),
        pltpu.VMEM((CH,), jnp.int32),
        pltpu.VMEM((CH, D), jnp.float32),
        pltpu.VMEM((CH, D), jnp.float32),
        pltpu.VMEM((CH,), jnp.float32),
        pltpu.VMEM((L,), jnp.float32),
        pltpu.VMEM((L * L,), jnp.float32),
        pltpu.VMEM((3 * L,), jnp.int32),
        pltpu.VMEM((2 * L,), jnp.float32),
        pltpu.SemaphoreType.DMA,
        pltpu.SemaphoreType.DMA,
    ],
    compiler_params=pltpu.CompilerParams(needs_layout_passes=False),
)


def _agg_body(vflat_hbm, src_hbm, dst_hbm, et_hbm, attn_hbm, mpart_hbm,
              out_hbm, den_hbm,
              m_loc, den_loc, tmp_v, red_v, dst_v, src_v, et_v, vidx,
              attn_v, ex_v, v_rows, w2, kb, vb, t_shared, m_sh, sem1, sem2):
    c = lax.axis_index("c")
    s = lax.axis_index("s")
    w = s * NC + c

    def initr_body(j, _):
        red_v[pl.ds(j * L, L)] = jnp.full((L,), _NEG_BIG, jnp.float32)
        return 0
    lax.fori_loop(0, NSL // L, initr_body, 0)

    def rrow_body(r, _):
        pltpu.sync_copy(mpart_hbm.at[r, pl.ds(s * NSL, NSL)], tmp_v)

        def rb(j, _):
            sl = pl.ds(j * L, L)
            red_v[sl] = jnp.maximum(red_v[sl], tmp_v[sl])
            return 0
        lax.fori_loop(0, NSL // L, rb, 0)
        return 0
    lax.fori_loop(0, NW, rrow_body, 0)
    pltpu.sync_copy(red_v, m_sh.at[pl.ds(s * NSL, NSL)])

    zero = jnp.zeros((L,), jnp.float32)

    def zw_body(r, _):
        for j in range(D // L):
            w2[r, pl.ds(j * L, L)] = zero
        return 0
    lax.fori_loop(0, CH, zw_body, 0)

    def zd_body(i, _):
        den_loc[pl.ds(i * L, L)] = zero
        return 0
    lax.fori_loop(0, N_PAD // L, zd_body, 0)

    kb[pl.ds(0, L)] = jnp.full((L,), -1, jnp.int32)
    kb[pl.ds(2 * L, L)] = jnp.full((L,), -2, jnp.int32)
    vb[pl.ds(0, L)] = zero

    rows0 = s * NSL
    for blk in range(NSL // CH):
        pltpu.sync_copy(w2, t_shared.at[pl.ds(rows0 + blk * CH, CH)])

    plsc.subcore_barrier()
    pltpu.sync_copy(m_sh, m_loc)

    nchunks = jnp.where(w < NCHUNKC - (NCHUNKC // NW) * NW,
                        NCHUNKC // NW + 1, NCHUNKC // NW)

    def chunk_body(i, _):
        base = (w + i * NW) * CHC
        pltpu.sync_copy(src_hbm.at[pl.ds(base, CHC)], src_v)
        pltpu.sync_copy(dst_hbm.at[pl.ds(base, CHC)], dst_v)
        pltpu.sync_copy(et_hbm.at[pl.ds(base, CHC)], et_v)
        pltpu.sync_copy(attn_hbm.at[pl.ds(base, CHC)], attn_v)

        def idx_body(j, _):
            sl = pl.ds(j * L, L)
            vidx[sl] = src_v[sl] + et_v[sl] * N
            return 0
        lax.fori_loop(0, CHC // L, idx_body, 0)

        cp = pltpu.async_copy(vflat_hbm.at[vidx], v_rows, sem1)

        def ex_body(j, _):
            sl = pl.ds(j * L, L)
            md = plsc.load_gather(m_loc, [dst_v[sl]])
            ex_v[sl] = jnp.exp(attn_v[sl] - md)
            return 0
        lax.fori_loop(0, CHC // L, ex_body, 0)

        cp.wait()

        def group_body(g, _):
            eb = g * L
            sl_g = pl.ds(eb, L)
            x16 = ex_v[sl_g]
            for el in range(L):
                x = x16[el]
                e = eb + el
                for j in range(D // L):
                    w2[e, pl.ds(j * L, L)] = v_rows[e, pl.ds(j * L, L)] * x
            dst16 = dst_v[sl_g]
            sk_sv = plsc.sort_key_val(dst16, x16)
            sk = sk_sv[0]
            sv = sk_sv[1]
            kb[pl.ds(L, L)] = sk
            vb[pl.ds(L, L)] = sv
            for shift in (1, 2, 4, 8):
                ks = kb[pl.ds(L - shift, L)]
                vs = vb[pl.ds(L - shift, L)]
                sv = sv + jnp.where(ks == sk, vs, 0.0)
                vb[pl.ds(L, L)] = sv
            nxt = kb[pl.ds(L + 1, L)]
            is_last = sk != nxt
            plsc.addupdate_scatter(den_loc, [sk], sv, mask=is_last)
            return 0
        lax.fori_loop(0, CHC // L, group_body, 0)

        pltpu.sync_copy(w2.at[pl.ds(0, CHC)], t_shared.at[dst_v], add=True)
        return 0
    lax.fori_loop(0, nchunks, chunk_body, 0)

    pltpu.sync_copy(den_loc, den_hbm.at[w])
    plsc.subcore_barrier()
    pltpu.sync_copy(t_shared.at[pl.ds(rows0, NSL)],
                    out_hbm.at[c, pl.ds(rows0, NSL)])


_agg_kernel = pl.kernel(
    _agg_body,
    out_type=[
        jax.ShapeDtypeStruct((NC, N_PAD, D), jnp.float32),
        jax.ShapeDtypeStruct((NW, N_PAD), jnp.float32),
    ],
    mesh=_MESH,
    scratch_types=[
        pltpu.VMEM((N_PAD,), jnp.float32),
        pltpu.VMEM((N_PAD,), jnp.float32),
        pltpu.VMEM((NSL,), jnp.float32),
        pltpu.VMEM((NSL,), jnp.float32),
        pltpu.VMEM((CHC,), jnp.int32),
        pltpu.VMEM((CHC,), jnp.int32),
        pltpu.VMEM((CHC,), jnp.int32),
        pltpu.VMEM((CHC,), jnp.int32),
        pltpu.VMEM((CHC,), jnp.float32),
        pltpu.VMEM((CHC,), jnp.float32),
        pltpu.VMEM((CHC, D), jnp.float32),
        pltpu.VMEM((CH, D), jnp.float32),
        pltpu.VMEM((3 * L,), jnp.int32),
        pltpu.VMEM((2 * L,), jnp.float32),
        pltpu.VMEM_SHARED((N_PAD, D), jnp.float32),
        pltpu.VMEM_SHARED((N_PAD,), jnp.float32),
        pltpu.SemaphoreType.DMA,
        pltpu.SemaphoreType.DMA,
    ],
    compiler_params=pltpu.CompilerParams(needs_layout_passes=False),
)


def _out_body(nt_ref, tp_ref, dp_ref, aw_ref, alpha_ref, out_ref):
    t = tp_ref[0] + tp_ref[1]
    dp = dp_ref[...]
    den = lax.dot_general(dp, jnp.ones((NW, 1), jnp.float32),
                          (((1,), (0,)), ((), ())),
                          preferred_element_type=jnp.float32)
    safe = den > 0
    x = jnp.where(safe, t / jnp.where(safe, den, 1.0), 0.0)
    nt = nt_ref[...]
    acc = jnp.zeros((x.shape[0], D), jnp.float32)
    alpha = jnp.zeros((x.shape[0], 1), jnp.float32)
    for tt in range(NT):
        m = (nt == tt).astype(jnp.float32)
        acc += m * jnp.dot(x, aw_ref[tt], preferred_element_type=jnp.float32)
        alpha += m * alpha_ref[tt]
    out_ref[...] = acc * alpha


def _final(nt2, t_part, den_part, aw, alpha8):
    grid = N // _BN
    return pl.pallas_call(
        _out_body,
        grid=(grid,),
        in_specs=[
            pl.BlockSpec((_BN, 1), lambda i: (i, 0)),
            pl.BlockSpec((NC, _BN, D), lambda i: (0, i, 0)),
            pl.BlockSpec((_BN, NW), lambda i: (i, 0)),
            pl.BlockSpec((NT, D, D), lambda i: (0, 0, 0)),
            pl.BlockSpec((8, 1), lambda i: (0, 0)),
        ],
        out_specs=pl.BlockSpec((_BN, D), lambda i: (i, 0)),
        out_shape=jax.ShapeDtypeStruct((N, D), jnp.float32),
    )(nt2, t_part, den_part, aw, alpha8)


def kernel(h, k_weight, q_weight, v_weight, a_weight, relation_pri,
           relation_att, relation_msg, skip, edge_index, node_type,
           edge_type):
    nt2 = node_type.reshape(N, 1)
    q_node, k_all, v_all = _proj(nt2, h, k_weight, q_weight, v_weight,
                                 relation_att, relation_msg)
    k_flat = k_all.reshape(NR * N, D)
    v_flat = v_all.reshape(NR * N, D)

    src = edge_index[0]
    dst = edge_index[1]
    pri16 = jnp.zeros((L,), jnp.float32).at[:NR].set(
        relation_pri.reshape(NR).astype(jnp.float32))

    attn, m_part = _attn_kernel(k_flat, q_node, src, dst, edge_type, pri16)
    t_part, den_part = _agg_kernel(v_flat, src, dst, edge_type, attn,
                                   m_part)
    den_t = den_part.T

    alpha8 = jnp.zeros((8, 1), jnp.float32).at[:NT, 0].set(
        jax.nn.sigmoid(skip.astype(jnp.float32)))
    return _final(nt2, t_part, den_t, a_weight, alpha8)


# speedup vs baseline: 9.0881x; 1.3437x over previous
import functools
import math

import jax
import jax.numpy as jnp
from jax import lax
from jax.experimental import pallas as pl
from jax.experimental.pallas import tpu as pltpu
from jax.experimental.pallas import tpu_sc as plsc

N = 10000
E = 160000
D = 128
NT = 4
NR = 4

NC = 2
NS = 16
NW = NC * NS
L = 16

CH = 128
NCHUNK = E // CH
CHC = 64
NCHUNKC = E // CHC
N_PAD = 10240
NSL = N_PAD // NS

_INV_SQRT_DK = 1.0 / math.sqrt(float(D))
_NEG_BIG = -3.0e38


_BN = 1000


def _proj_body(nt_ref, h_ref, kw_ref, qw_ref, vw_ref, att_ref, msg_ref,
               q_out, k_out, v_out):
    h = h_ref[...]
    nt = nt_ref[...]
    kn = jnp.zeros_like(h)
    qn = jnp.zeros_like(h)
    vn = jnp.zeros_like(h)
    for t in range(NT):
        m = (nt == t).astype(jnp.float32)
        kn += m * jnp.dot(h, kw_ref[t], preferred_element_type=jnp.float32)
        qn += m * jnp.dot(h, qw_ref[t], preferred_element_type=jnp.float32)
        vn += m * jnp.dot(h, vw_ref[t], preferred_element_type=jnp.float32)
    q_out[...] = qn
    for r in range(NR):
        k_out[r] = jnp.dot(kn, att_ref[r], preferred_element_type=jnp.float32)
        v_out[r] = jnp.dot(vn, msg_ref[r], preferred_element_type=jnp.float32)


def _proj(nt2, h, kw, qw, vw, att, msg):
    grid = N // _BN
    return pl.pallas_call(
        _proj_body,
        grid=(grid,),
        in_specs=[
            pl.BlockSpec((_BN, 1), lambda i: (i, 0)),
            pl.BlockSpec((_BN, D), lambda i: (i, 0)),
            pl.BlockSpec((NT, D, D), lambda i: (0, 0, 0)),
            pl.BlockSpec((NT, D, D), lambda i: (0, 0, 0)),
            pl.BlockSpec((NT, D, D), lambda i: (0, 0, 0)),
            pl.BlockSpec((NR, D, D), lambda i: (0, 0, 0)),
            pl.BlockSpec((NR, D, D), lambda i: (0, 0, 0)),
        ],
        out_specs=[
            pl.BlockSpec((_BN, D), lambda i: (i, 0)),
            pl.BlockSpec((NR, _BN, D), lambda i: (0, i, 0)),
            pl.BlockSpec((NR, _BN, D), lambda i: (0, i, 0)),
        ],
        out_shape=[
            jax.ShapeDtypeStruct((N, D), jnp.float32),
            jax.ShapeDtypeStruct((NR, N, D), jnp.float32),
            jax.ShapeDtypeStruct((NR, N, D), jnp.float32),
        ],
    )(nt2, h, kw, qw, vw, att, msg)


_MESH = plsc.VectorSubcoreMesh(core_axis_name="c", subcore_axis_name="s")


def _attn_body(kflat_hbm, qnode_hbm, src_hbm, dst_hbm, et_hbm, pri_hbm,
               attn_hbm, mpart_hbm,
               m_loc, kidx0, kidx1, dst_v0, dst_v1, src_v0, src_v1,
               et_v0, et_v1, k_rows0, k_rows1, q_rows0, q_rows1,
               attn_v0, attn_v1, pri_v, pbuf, kb, vb,
               semk0, semk1, semq0, semq1):
    c = lax.axis_index("c")
    s = lax.axis_index("s")
    w = s * NC + c
    lane = lax.iota(jnp.int32, L)
    tcol = lane * L
    zero = jnp.zeros((L,), jnp.float32)

    kidxs = (kidx0, kidx1)
    dst_vs = (dst_v0, dst_v1)
    src_vs = (src_v0, src_v1)
    et_vs = (et_v0, et_v1)
    k_rowss = (k_rows0, k_rows1)
    q_rowss = (q_rows0, q_rows1)
    attn_vs = (attn_v0, attn_v1)
    semks = (semk0, semk1)
    semqs = (semq0, semq1)

    def init_body(i, _):
        m_loc[pl.ds(i * L, L)] = jnp.full((L,), _NEG_BIG, jnp.float32)
        return 0
    lax.fori_loop(0, N_PAD // L, init_body, 0)

    pltpu.sync_copy(pri_hbm, pri_v)
    kb[pl.ds(0, L)] = jnp.full((L,), -1, jnp.int32)
    kb[pl.ds(2 * L, L)] = jnp.full((L,), -2, jnp.int32)
    vb[pl.ds(0, L)] = jnp.full((L,), _NEG_BIG, jnp.float32)

    NPW_A = (NCHUNK + NW - 1) // NW
    NH_A = NPW_A // 2

    def fire(pi, j):
        cid = lax.rem(w + j * NW, NCHUNK)
        base = cid * CH
        pltpu.sync_copy(src_hbm.at[pl.ds(base, CH)], src_vs[pi])
        pltpu.sync_copy(dst_hbm.at[pl.ds(base, CH)], dst_vs[pi])
        pltpu.sync_copy(et_hbm.at[pl.ds(base, CH)], et_vs[pi])

        def idx_body(jj, _):
            sl = pl.ds(jj * L, L)
            kidxs[pi][sl] = src_vs[pi][sl] + et_vs[pi][sl] * N
            return 0
        lax.fori_loop(0, CH // L, idx_body, 0)
        pltpu.async_copy(kflat_hbm.at[kidxs[pi]], k_rowss[pi], semks[pi])
        pltpu.async_copy(qnode_hbm.at[dst_vs[pi]], q_rowss[pi], semqs[pi])

    def compute(pi, j):
        cid = lax.rem(w + j * NW, NCHUNK)
        base = cid * CH
        pltpu.make_async_copy(kflat_hbm.at[kidxs[pi]], k_rowss[pi],
                              semks[pi]).wait()
        pltpu.make_async_copy(qnode_hbm.at[dst_vs[pi]], q_rowss[pi],
                              semqs[pi]).wait()
        k_rows = k_rowss[pi]
        q_rows = q_rowss[pi]
        attn_v = attn_vs[pi]
        dst_v = dst_vs[pi]
        et_v = et_vs[pi]

        def group_body(g, _):
            eb = g * L
            sl_g = pl.ds(eb, L)
            for el in range(L):
                e = eb + el
                p = zero
                for jj in range(D // L):
                    kk = k_rows[e, pl.ds(jj * L, L)]
                    qq = q_rows[e, pl.ds(jj * L, L)]
                    p = p + kk * qq
                pbuf[pl.ds(el * L, L)] = p
            acc = zero
            for l in range(L):
                acc = acc + plsc.load_gather(pbuf, [tcol + l])
            et16 = et_v[sl_g]
            pri16 = plsc.load_gather(pri_v, [et16])
            attn16 = acc * (pri16 * _INV_SQRT_DK)
            attn_v[sl_g] = attn16
            dst16 = dst_v[sl_g]
            sk = plsc.sort_key_val(dst16, attn16)
            sv = sk[1]
            sk = sk[0]
            kb[pl.ds(L, L)] = sk
            vb[pl.ds(L, L)] = sv
            for shift in (1, 2, 4, 8):
                ks = kb[pl.ds(L - shift, L)]
                vs = vb[pl.ds(L - shift, L)]
                sv = jnp.maximum(sv, jnp.where(ks == sk, vs, _NEG_BIG))
                vb[pl.ds(L, L)] = sv
            nxt = kb[pl.ds(L + 1, L)]
            is_last = sk != nxt
            cur = plsc.load_gather(m_loc, [sk])
            plsc.store_scatter(m_loc, [sk], jnp.maximum(cur, sv),
                               mask=is_last)
            return 0
        lax.fori_loop(0, CH // L, group_body, 0)

        pltpu.sync_copy(attn_v, attn_hbm.at[pl.ds(base, CH)])

    fire(0, 0)

    def pair_body(i, _):
        fire(1, 2 * i + 1)
        compute(0, 2 * i)

        @pl.when(i < NH_A - 1)
        def _():
            fire(0, 2 * i + 2)
        compute(1, 2 * i + 1)
        return 0
    lax.fori_loop(0, NH_A, pair_body, 0)

    pltpu.sync_copy(m_loc, mpart_hbm.at[w])


_attn_kernel = pl.kernel(
    _attn_body,
    out_type=[
        jax.ShapeDtypeStruct((E,), jnp.float32),
        jax.ShapeDtypeStruct((NW, N_PAD), jnp.float32),
    ],
    mesh=_MESH,
    scratch_types=[
        pltpu.VMEM((N_PAD,), jnp.float32),
        pltpu.VMEM((CH,), jnp.int32),
        pltpu.VMEM((CH,), jnp.int32),
        pltpu.VMEM((CH,), jnp.int32),
        pltpu.VMEM((CH,), jnp.int32),
        pltpu.VMEM((CH,), jnp.int32),
        pltpu.VMEM((CH,), jnp.int32),
        pltpu.VMEM((CH,), jnp.int32),
        pltpu.VMEM((CH,), jnp.int32),
        pltpu.VMEM((CH, D), jnp.float32),
        pltpu.VMEM((CH, D), jnp.float32),
        pltpu.VMEM((CH, D), jnp.float32),
        pltpu.VMEM((CH, D), jnp.float32),
        pltpu.VMEM((CH,), jnp.float32),
        pltpu.VMEM((CH,), jnp.float32),
        pltpu.VMEM((L,), jnp.float32),
        pltpu.VMEM((L * L,), jnp.float32),
        pltpu.VMEM((3 * L,), jnp.int32),
        pltpu.VMEM((2 * L,), jnp.float32),
        pltpu.SemaphoreType.DMA,
        pltpu.SemaphoreType.DMA,
        pltpu.SemaphoreType.DMA,
        pltpu.SemaphoreType.DMA,
    ],
    compiler_params=pltpu.CompilerParams(needs_layout_passes=False),
)


def _agg_body(vflat_hbm, src_hbm, dst_hbm, et_hbm, attn_hbm, mpart_hbm,
              out_hbm, den_hbm,
              m_loc, den_loc, tmp_v, red_v,
              dst_v0, dst_v1, src_v0, src_v1, et_v0, et_v1, vidx0, vidx1,
              attn_v0, attn_v1, ex_v0, ex_v1, v_rows0, v_rows1, w2,
              dsc0, dsc1, kb, vb, t_shared, m_sh,
              semv0, semv1, semsc):
    c = lax.axis_index("c")
    s = lax.axis_index("s")
    w = s * NC + c

    dst_vs = (dst_v0, dst_v1)
    src_vs = (src_v0, src_v1)
    et_vs = (et_v0, et_v1)
    vidxs = (vidx0, vidx1)
    attn_vs = (attn_v0, attn_v1)
    ex_vs = (ex_v0, ex_v1)
    v_rowss = (v_rows0, v_rows1)
    dscs = (dsc0, dsc1)
    semvs = (semv0, semv1)

    def initr_body(j, _):
        red_v[pl.ds(j * L, L)] = jnp.full((L,), _NEG_BIG, jnp.float32)
        return 0
    lax.fori_loop(0, NSL // L, initr_body, 0)

    def rrow_body(r, _):
        pltpu.sync_copy(mpart_hbm.at[r, pl.ds(s * NSL, NSL)], tmp_v)

        def rb(j, _):
            sl = pl.ds(j * L, L)
            red_v[sl] = jnp.maximum(red_v[sl], tmp_v[sl])
            return 0
        lax.fori_loop(0, NSL // L, rb, 0)
        return 0
    lax.fori_loop(0, NW, rrow_body, 0)
    pltpu.sync_copy(red_v, m_sh.at[pl.ds(s * NSL, NSL)])

    zero = jnp.zeros((L,), jnp.float32)

    def zw_body(r, _):
        for j in range(D // L):
            w2[r, pl.ds(j * L, L)] = zero
        return 0
    lax.fori_loop(0, CHC, zw_body, 0)

    def zd_body(i, _):
        den_loc[pl.ds(i * L, L)] = zero
        return 0
    lax.fori_loop(0, N_PAD // L, zd_body, 0)

    kb[pl.ds(0, L)] = jnp.full((L,), -1, jnp.int32)
    kb[pl.ds(2 * L, L)] = jnp.full((L,), -2, jnp.int32)
    vb[pl.ds(0, L)] = zero

    rows0 = s * NSL
    for blk in range(NSL // CHC):
        pltpu.sync_copy(w2, t_shared.at[pl.ds(rows0 + blk * CHC, CHC)])

    plsc.subcore_barrier()
    pltpu.sync_copy(m_sh, m_loc)

    nchunks = jnp.where(w < NCHUNKC - (NCHUNKC // NW) * NW,
                        NCHUNKC // NW + 1, NCHUNKC // NW)
    NH_C = (NCHUNKC // NW + 2) // 2

    def fire(pi, j):
        base = (w + j * NW) * CHC
        pltpu.sync_copy(src_hbm.at[pl.ds(base, CHC)], src_vs[pi])
        pltpu.sync_copy(dst_hbm.at[pl.ds(base, CHC)], dst_vs[pi])
        pltpu.sync_copy(et_hbm.at[pl.ds(base, CHC)], et_vs[pi])
        pltpu.sync_copy(attn_hbm.at[pl.ds(base, CHC)], attn_vs[pi])

        def idx_body(jj, _):
            sl = pl.ds(jj * L, L)
            vidxs[pi][sl] = src_vs[pi][sl] + et_vs[pi][sl] * N
            return 0
        lax.fori_loop(0, CHC // L, idx_body, 0)
        pltpu.async_copy(vflat_hbm.at[vidxs[pi]], v_rowss[pi], semvs[pi])

    def compute(pi, j):
        @pl.when(j >= 1)
        def _():
            pltpu.make_async_copy(w2, t_shared.at[dscs[1 - pi]],
                                  semsc).wait()
        pltpu.make_async_copy(vflat_hbm.at[vidxs[pi]], v_rowss[pi],
                              semvs[pi]).wait()
        ex_v = ex_vs[pi]
        attn_v = attn_vs[pi]
        dst_v = dst_vs[pi]
        v_rows = v_rowss[pi]

        def ex_body(jj, _):
            sl = pl.ds(jj * L, L)
            md = plsc.load_gather(m_loc, [dst_v[sl]])
            ex_v[sl] = jnp.exp(attn_v[sl] - md)
            return 0
        lax.fori_loop(0, CHC // L, ex_body, 0)

        def group_body(g, _):
            eb = g * L
            sl_g = pl.ds(eb, L)
            x16 = ex_v[sl_g]
            for el in range(L):
                x = x16[el]
                e = eb + el
                for jj in range(D // L):
                    w2[e, pl.ds(jj * L, L)] = v_rows[e, pl.ds(jj * L, L)] * x
            dst16 = dst_v[sl_g]
            sk_sv = plsc.sort_key_val(dst16, x16)
            sk = sk_sv[0]
            sv = sk_sv[1]
            kb[pl.ds(L, L)] = sk
            vb[pl.ds(L, L)] = sv
            for shift in (1, 2, 4, 8):
                ks = kb[pl.ds(L - shift, L)]
                vs = vb[pl.ds(L - shift, L)]
                sv = sv + jnp.where(ks == sk, vs, 0.0)
                vb[pl.ds(L, L)] = sv
            nxt = kb[pl.ds(L + 1, L)]
            is_last = sk != nxt
            plsc.addupdate_scatter(den_loc, [sk], sv, mask=is_last)
            return 0
        lax.fori_loop(0, CHC // L, group_body, 0)

        def cp_body(jj, _):
            sl = pl.ds(jj * L, L)
            dscs[pi][sl] = dst_v[sl]
            return 0
        lax.fori_loop(0, CHC // L, cp_body, 0)
        pltpu.async_copy(w2, t_shared.at[dscs[pi]], semsc, add=True)

    fire(0, 0)

    def pair_body(i, _):
        a = 2 * i
        b = 2 * i + 1

        @pl.when(b < nchunks)
        def _():
            fire(1, b)

        compute(0, a)

        @pl.when(a + 2 < nchunks)
        def _():
            fire(0, a + 2)

        @pl.when(b < nchunks)
        def _():
            compute(1, b)
        return 0
    lax.fori_loop(0, (nchunks + 1) // 2, pair_body, 0)

    p_last = lax.rem(nchunks - 1, 2)

    @pl.when(p_last == 0)
    def _():
        pltpu.make_async_copy(w2, t_shared.at[dsc0], semsc).wait()

    @pl.when(p_last == 1)
    def _():
        pltpu.make_async_copy(w2, t_shared.at[dsc1], semsc).wait()

    pltpu.sync_copy(den_loc, den_hbm.at[w])
    plsc.subcore_barrier()
    pltpu.sync_copy(t_shared.at[pl.ds(rows0, NSL)],
                    out_hbm.at[c, pl.ds(rows0, NSL)])


_agg_kernel = pl.kernel(
    _agg_body,
    out_type=[
        jax.ShapeDtypeStruct((NC, N_PAD, D), jnp.float32),
        jax.ShapeDtypeStruct((NW, N_PAD), jnp.float32),
    ],
    mesh=_MESH,
    scratch_types=[
        pltpu.VMEM((N_PAD,), jnp.float32),
        pltpu.VMEM((N_PAD,), jnp.float32),
        pltpu.VMEM((NSL,), jnp.float32),
        pltpu.VMEM((NSL,), jnp.float32),
        pltpu.VMEM((CHC,), jnp.int32),
        pltpu.VMEM((CHC,), jnp.int32),
        pltpu.VMEM((CHC,), jnp.int32),
        pltpu.VMEM((CHC,), jnp.int32),
        pltpu.VMEM((CHC,), jnp.int32),
        pltpu.VMEM((CHC,), jnp.int32),
        pltpu.VMEM((CHC,), jnp.int32),
        pltpu.VMEM((CHC,), jnp.int32),
        pltpu.VMEM((CHC,), jnp.float32),
        pltpu.VMEM((CHC,), jnp.float32),
        pltpu.VMEM((CHC,), jnp.float32),
        pltpu.VMEM((CHC,), jnp.float32),
        pltpu.VMEM((CHC, D), jnp.float32),
        pltpu.VMEM((CHC, D), jnp.float32),
        pltpu.VMEM((CHC, D), jnp.float32),
        pltpu.VMEM((CHC,), jnp.int32),
        pltpu.VMEM((CHC,), jnp.int32),
        pltpu.VMEM((3 * L,), jnp.int32),
        pltpu.VMEM((2 * L,), jnp.float32),
        pltpu.VMEM_SHARED((N_PAD, D), jnp.float32),
        pltpu.VMEM_SHARED((N_PAD,), jnp.float32),
        pltpu.SemaphoreType.DMA,
        pltpu.SemaphoreType.DMA,
        pltpu.SemaphoreType.DMA,
    ],
    compiler_params=pltpu.CompilerParams(needs_layout_passes=False),
)


def _out_body(nt_ref, tp_ref, dp_ref, aw_ref, alpha_ref, out_ref):
    t = tp_ref[0] + tp_ref[1]
    dp = dp_ref[...]
    den = lax.dot_general(dp, jnp.ones((NW, 1), jnp.float32),
                          (((1,), (0,)), ((), ())),
                          preferred_element_type=jnp.float32)
    safe = den > 0
    x = jnp.where(safe, t / jnp.where(safe, den, 1.0), 0.0)
    nt = nt_ref[...]
    acc = jnp.zeros((x.shape[0], D), jnp.float32)
    alpha = jnp.zeros((x.shape[0], 1), jnp.float32)
    for tt in range(NT):
        m = (nt == tt).astype(jnp.float32)
        acc += m * jnp.dot(x, aw_ref[tt], preferred_element_type=jnp.float32)
        alpha += m * alpha_ref[tt]
    out_ref[...] = acc * alpha


def _final(nt2, t_part, den_part, aw, alpha8):
    grid = N // _BN
    return pl.pallas_call(
        _out_body,
        grid=(grid,),
        in_specs=[
            pl.BlockSpec((_BN, 1), lambda i: (i, 0)),
            pl.BlockSpec((NC, _BN, D), lambda i: (0, i, 0)),
            pl.BlockSpec((_BN, NW), lambda i: (i, 0)),
            pl.BlockSpec((NT, D, D), lambda i: (0, 0, 0)),
            pl.BlockSpec((8, 1), lambda i: (0, 0)),
        ],
        out_specs=pl.BlockSpec((_BN, D), lambda i: (i, 0)),
        out_shape=jax.ShapeDtypeStruct((N, D), jnp.float32),
    )(nt2, t_part, den_part, aw, alpha8)


def kernel(h, k_weight, q_weight, v_weight, a_weight, relation_pri,
           relation_att, relation_msg, skip, edge_index, node_type,
           edge_type):
    nt2 = node_type.reshape(N, 1)
    q_node, k_all, v_all = _proj(nt2, h, k_weight, q_weight, v_weight,
                                 relation_att, relation_msg)
    k_flat = k_all.reshape(NR * N, D)
    v_flat = v_all.reshape(NR * N, D)

    src = edge_index[0]
    dst = edge_index[1]
    pri16 = jnp.zeros((L,), jnp.float32).at[:NR].set(
        relation_pri.reshape(NR).astype(jnp.float32))

    attn, m_part = _attn_kernel(k_flat, q_node, src, dst, edge_type, pri16)
    t_part, den_part = _agg_kernel(v_flat, src, dst, edge_type, attn,
                                   m_part)
    den_t = den_part.T

    alpha8 = jnp.zeros((8, 1), jnp.float32).at[:NT, 0].set(
        jax.nn.sigmoid(skip.astype(jnp.float32)))
    return _final(nt2, t_part, den_t, a_weight, alpha8)


# speedup vs baseline: 12.2121x; 1.3438x over previous
import functools
import math

import jax
import jax.numpy as jnp
from jax import lax
from jax.experimental import pallas as pl
from jax.experimental.pallas import tpu as pltpu
from jax.experimental.pallas import tpu_sc as plsc

N = 10000
E = 160000
D = 128
NT = 4
NR = 4

NC = 2
NS = 16
NW = NC * NS
L = 16

CH = 128
NCHUNK = E // CH
CHC = 64
NCHUNKC = E // CHC
N_PAD = 10240
NSL = N_PAD // NS

_INV_SQRT_DK = 1.0 / math.sqrt(float(D))
_NEG_BIG = -3.0e38


_BN = 1000


def _proj_body(nt_ref, h_ref, kw_ref, qw_ref, vw_ref, att_ref, msg_ref,
               q_out, k_out, v_out):
    h = h_ref[...]
    nt = nt_ref[...]
    kn = jnp.zeros_like(h)
    qn = jnp.zeros_like(h)
    vn = jnp.zeros_like(h)
    for t in range(NT):
        m = (nt == t).astype(jnp.float32)
        kn += m * jnp.dot(h, kw_ref[t], preferred_element_type=jnp.float32)
        qn += m * jnp.dot(h, qw_ref[t], preferred_element_type=jnp.float32)
        vn += m * jnp.dot(h, vw_ref[t], preferred_element_type=jnp.float32)
    q_out[...] = qn
    for r in range(NR):
        k_out[r] = jnp.dot(kn, att_ref[r], preferred_element_type=jnp.float32)
        v_out[r] = jnp.dot(vn, msg_ref[r], preferred_element_type=jnp.float32)


def _proj(nt2, h, kw, qw, vw, att, msg):
    grid = N // _BN
    return pl.pallas_call(
        _proj_body,
        grid=(grid,),
        in_specs=[
            pl.BlockSpec((_BN, 1), lambda i: (i, 0)),
            pl.BlockSpec((_BN, D), lambda i: (i, 0)),
            pl.BlockSpec((NT, D, D), lambda i: (0, 0, 0)),
            pl.BlockSpec((NT, D, D), lambda i: (0, 0, 0)),
            pl.BlockSpec((NT, D, D), lambda i: (0, 0, 0)),
            pl.BlockSpec((NR, D, D), lambda i: (0, 0, 0)),
            pl.BlockSpec((NR, D, D), lambda i: (0, 0, 0)),
        ],
        out_specs=[
            pl.BlockSpec((_BN, D), lambda i: (i, 0)),
            pl.BlockSpec((NR, _BN, D), lambda i: (0, i, 0)),
            pl.BlockSpec((NR, _BN, D), lambda i: (0, i, 0)),
        ],
        out_shape=[
            jax.ShapeDtypeStruct((N, D), jnp.float32),
            jax.ShapeDtypeStruct((NR, N, D), jnp.float32),
            jax.ShapeDtypeStruct((NR, N, D), jnp.float32),
        ],
    )(nt2, h, kw, qw, vw, att, msg)


_MESH = plsc.VectorSubcoreMesh(core_axis_name="c", subcore_axis_name="s")


def _attn_body(kflat_hbm, qnode_hbm, aux_hbm, pri_hbm,
               attn_hbm, mpart_hbm,
               m_loc, kidx0, kidx1, aux_v0, aux_v1,
               k_rows0, k_rows1, q_rows0, q_rows1,
               attn_v0, attn_v1, pri_v, pbuf, kb, vb,
               semk0, semk1, semq0, semq1):
    c = lax.axis_index("c")
    s = lax.axis_index("s")
    w = s * NC + c
    lane = lax.iota(jnp.int32, L)
    tcol = lane * L
    zero = jnp.zeros((L,), jnp.float32)

    kidxs = (kidx0, kidx1)
    aux_vs = (aux_v0, aux_v1)
    k_rowss = (k_rows0, k_rows1)
    q_rowss = (q_rows0, q_rows1)
    attn_vs = (attn_v0, attn_v1)
    semks = (semk0, semk1)
    semqs = (semq0, semq1)

    def init_body(i, _):
        m_loc[pl.ds(i * L, L)] = jnp.full((L,), _NEG_BIG, jnp.float32)
        return 0
    lax.fori_loop(0, N_PAD // L, init_body, 0)

    pltpu.sync_copy(pri_hbm, pri_v)
    kb[pl.ds(0, L)] = jnp.full((L,), -1, jnp.int32)
    kb[pl.ds(2 * L, L)] = jnp.full((L,), -2, jnp.int32)
    vb[pl.ds(0, L)] = jnp.full((L,), _NEG_BIG, jnp.float32)

    NPW_A = (NCHUNK + NW - 1) // NW
    NH_A = NPW_A // 2

    def fire(pi, j):
        cid = lax.rem(w + j * NW, NCHUNK)
        pltpu.sync_copy(aux_hbm.at[cid], aux_vs[pi])

        def idx_body(jj, _):
            sl = pl.ds(jj * L, L)
            kidxs[pi][sl] = (aux_vs[pi][pl.ds(jj * L, L)]
                             + aux_vs[pi][pl.ds(2 * CH + jj * L, L)] * N)
            return 0
        lax.fori_loop(0, CH // L, idx_body, 0)
        pltpu.async_copy(kflat_hbm.at[kidxs[pi]], k_rowss[pi], semks[pi])
        pltpu.async_copy(qnode_hbm.at[aux_vs[pi].at[pl.ds(CH, CH)]],
                         q_rowss[pi], semqs[pi])

    def compute(pi, j):
        cid = lax.rem(w + j * NW, NCHUNK)
        base = cid * CH
        pltpu.make_async_copy(kflat_hbm.at[kidxs[pi]], k_rowss[pi],
                              semks[pi]).wait()
        pltpu.make_async_copy(qnode_hbm.at[aux_vs[pi].at[pl.ds(CH, CH)]],
                              q_rowss[pi], semqs[pi]).wait()
        k_rows = k_rowss[pi]
        q_rows = q_rowss[pi]
        attn_v = attn_vs[pi]
        aux_v = aux_vs[pi]

        def group_body(g, _):
            eb = g * L
            sl_g = pl.ds(eb, L)
            for el in range(L):
                e = eb + el
                p = zero
                for jj in range(D // L):
                    kk = k_rows[e, pl.ds(jj * L, L)]
                    qq = q_rows[e, pl.ds(jj * L, L)]
                    p = p + kk * qq
                pbuf[pl.ds(el * L, L)] = p
            acc = zero
            for l in range(L):
                acc = acc + plsc.load_gather(pbuf, [tcol + l])
            et16 = aux_v[pl.ds(2 * CH + eb, L)]
            pri16 = plsc.load_gather(pri_v, [et16])
            attn16 = acc * (pri16 * _INV_SQRT_DK)
            attn_v[sl_g] = attn16
            dst16 = aux_v[pl.ds(CH + eb, L)]
            sk = plsc.sort_key_val(dst16, attn16)
            sv = sk[1]
            sk = sk[0]
            kb[pl.ds(L, L)] = sk
            vb[pl.ds(L, L)] = sv
            for shift in (1, 2, 4, 8):
                ks = kb[pl.ds(L - shift, L)]
                vs = vb[pl.ds(L - shift, L)]
                sv = jnp.maximum(sv, jnp.where(ks == sk, vs, _NEG_BIG))
                vb[pl.ds(L, L)] = sv
            nxt = kb[pl.ds(L + 1, L)]
            is_last = sk != nxt
            cur = plsc.load_gather(m_loc, [sk])
            plsc.store_scatter(m_loc, [sk], jnp.maximum(cur, sv),
                               mask=is_last)
            return 0
        lax.fori_loop(0, CH // L, group_body, 0)

        pltpu.sync_copy(attn_v, attn_hbm.at[pl.ds(base, CH)])

    fire(0, 0)

    def pair_body(i, _):
        fire(1, 2 * i + 1)
        compute(0, 2 * i)

        @pl.when(i < NH_A - 1)
        def _():
            fire(0, 2 * i + 2)
        compute(1, 2 * i + 1)
        return 0
    lax.fori_loop(0, NH_A, pair_body, 0)

    pltpu.sync_copy(m_loc, mpart_hbm.at[w])


_attn_kernel = pl.kernel(
    _attn_body,
    out_type=[
        jax.ShapeDtypeStruct((E,), jnp.float32),
        jax.ShapeDtypeStruct((NW, N_PAD), jnp.float32),
    ],
    mesh=_MESH,
    scratch_types=[
        pltpu.VMEM((N_PAD,), jnp.float32),
        pltpu.VMEM((CH,), jnp.int32),
        pltpu.VMEM((CH,), jnp.int32),
        pltpu.VMEM((3 * CH,), jnp.int32),
        pltpu.VMEM((3 * CH,), jnp.int32),
        pltpu.VMEM((CH, D), jnp.float32),
        pltpu.VMEM((CH, D), jnp.float32),
        pltpu.VMEM((CH, D), jnp.float32),
        pltpu.VMEM((CH, D), jnp.float32),
        pltpu.VMEM((CH,), jnp.float32),
        pltpu.VMEM((CH,), jnp.float32),
        pltpu.VMEM((L,), jnp.float32),
        pltpu.VMEM((L * L,), jnp.float32),
        pltpu.VMEM((3 * L,), jnp.int32),
        pltpu.VMEM((2 * L,), jnp.float32),
        pltpu.SemaphoreType.DMA,
        pltpu.SemaphoreType.DMA,
        pltpu.SemaphoreType.DMA,
        pltpu.SemaphoreType.DMA,
    ],
    compiler_params=pltpu.CompilerParams(needs_layout_passes=False),
)


def _agg_body(vflat_hbm, aux_hbm, mpart_hbm,
              out_hbm, den_hbm,
              m_loc, den_loc, tmp_v, red_v,
              aux_v0, aux_v1, vidx0, vidx1,
              ex_v0, ex_v1, v_rows0, v_rows1, w2,
              dsc0, dsc1, kb, vb, t_shared, m_sh,
              semv0, semv1, semsc):
    c = lax.axis_index("c")
    s = lax.axis_index("s")
    w = s * NC + c

    aux_vs = (aux_v0, aux_v1)
    vidxs = (vidx0, vidx1)
    ex_vs = (ex_v0, ex_v1)
    v_rowss = (v_rows0, v_rows1)
    dscs = (dsc0, dsc1)
    semvs = (semv0, semv1)

    def initr_body(j, _):
        red_v[pl.ds(j * L, L)] = jnp.full((L,), _NEG_BIG, jnp.float32)
        return 0
    lax.fori_loop(0, NSL // L, initr_body, 0)

    def rrow_body(r, _):
        pltpu.sync_copy(mpart_hbm.at[r, pl.ds(s * NSL, NSL)], tmp_v)

        def rb(j, _):
            sl = pl.ds(j * L, L)
            red_v[sl] = jnp.maximum(red_v[sl], tmp_v[sl])
            return 0
        lax.fori_loop(0, NSL // L, rb, 0)
        return 0
    lax.fori_loop(0, NW, rrow_body, 0)
    pltpu.sync_copy(red_v, m_sh.at[pl.ds(s * NSL, NSL)])

    zero = jnp.zeros((L,), jnp.float32)

    def zw_body(r, _):
        for j in range(D // L):
            w2[r, pl.ds(j * L, L)] = zero
        return 0
    lax.fori_loop(0, CHC, zw_body, 0)

    def zd_body(i, _):
        den_loc[pl.ds(i * L, L)] = zero
        return 0
    lax.fori_loop(0, N_PAD // L, zd_body, 0)

    kb[pl.ds(0, L)] = jnp.full((L,), -1, jnp.int32)
    kb[pl.ds(2 * L, L)] = jnp.full((L,), -2, jnp.int32)
    vb[pl.ds(0, L)] = zero

    rows0 = s * NSL
    for blk in range(NSL // CHC):
        pltpu.sync_copy(w2, t_shared.at[pl.ds(rows0 + blk * CHC, CHC)])

    plsc.subcore_barrier()
    pltpu.sync_copy(m_sh, m_loc)

    nchunks = jnp.where(w < NCHUNKC - (NCHUNKC // NW) * NW,
                        NCHUNKC // NW + 1, NCHUNKC // NW)
    NH_C = (NCHUNKC // NW + 2) // 2

    def fire(pi, j):
        cid = w + j * NW
        pltpu.sync_copy(aux_hbm.at[cid], aux_vs[pi])

        def idx_body(jj, _):
            sl = pl.ds(jj * L, L)
            vidxs[pi][sl] = (aux_vs[pi][pl.ds(jj * L, L)]
                             + aux_vs[pi][pl.ds(2 * CHC + jj * L, L)] * N)
            return 0
        lax.fori_loop(0, CHC // L, idx_body, 0)
        pltpu.async_copy(vflat_hbm.at[vidxs[pi]], v_rowss[pi], semvs[pi])

    def compute(pi, j):
        @pl.when(j >= 1)
        def _():
            pltpu.make_async_copy(w2, t_shared.at[dscs[1 - pi]],
                                  semsc).wait()
        pltpu.make_async_copy(vflat_hbm.at[vidxs[pi]], v_rowss[pi],
                              semvs[pi]).wait()
        ex_v = ex_vs[pi]
        aux_v = aux_vs[pi]
        v_rows = v_rowss[pi]

        def ex_body(jj, _):
            md = plsc.load_gather(m_loc, [aux_v[pl.ds(CHC + jj * L, L)]])
            at16 = plsc.bitcast(aux_v[pl.ds(3 * CHC + jj * L, L)],
                                jnp.float32)
            ex_v[pl.ds(jj * L, L)] = jnp.exp(at16 - md)
            return 0
        lax.fori_loop(0, CHC // L, ex_body, 0)

        def group_body(g, _):
            eb = g * L
            sl_g = pl.ds(eb, L)
            x16 = ex_v[sl_g]
            for el in range(L):
                x = x16[el]
                e = eb + el
                for jj in range(D // L):
                    w2[e, pl.ds(jj * L, L)] = v_rows[e, pl.ds(jj * L, L)] * x
            dst16 = aux_v[pl.ds(CHC + eb, L)]
            sk_sv = plsc.sort_key_val(dst16, x16)
            sk = sk_sv[0]
            sv = sk_sv[1]
            kb[pl.ds(L, L)] = sk
            vb[pl.ds(L, L)] = sv
            for shift in (1, 2, 4, 8):
                ks = kb[pl.ds(L - shift, L)]
                vs = vb[pl.ds(L - shift, L)]
                sv = sv + jnp.where(ks == sk, vs, 0.0)
                vb[pl.ds(L, L)] = sv
            nxt = kb[pl.ds(L + 1, L)]
            is_last = sk != nxt
            plsc.addupdate_scatter(den_loc, [sk], sv, mask=is_last)
            return 0
        lax.fori_loop(0, CHC // L, group_body, 0)

        def cp_body(jj, _):
            sl = pl.ds(jj * L, L)
            dscs[pi][sl] = aux_v[pl.ds(CHC + jj * L, L)]
            return 0
        lax.fori_loop(0, CHC // L, cp_body, 0)
        pltpu.async_copy(w2, t_shared.at[dscs[pi]], semsc, add=True)

    fire(0, 0)

    def pair_body(i, _):
        a = 2 * i
        b = 2 * i + 1

        @pl.when(b < nchunks)
        def _():
            fire(1, b)

        compute(0, a)

        @pl.when(a + 2 < nchunks)
        def _():
            fire(0, a + 2)

        @pl.when(b < nchunks)
        def _():
            compute(1, b)
        return 0
    lax.fori_loop(0, (nchunks + 1) // 2, pair_body, 0)

    p_last = lax.rem(nchunks - 1, 2)

    @pl.when(p_last == 0)
    def _():
        pltpu.make_async_copy(w2, t_shared.at[dsc0], semsc).wait()

    @pl.when(p_last == 1)
    def _():
        pltpu.make_async_copy(w2, t_shared.at[dsc1], semsc).wait()

    pltpu.sync_copy(den_loc, den_hbm.at[w])
    plsc.subcore_barrier()
    pltpu.sync_copy(t_shared.at[pl.ds(rows0, NSL)],
                    out_hbm.at[c, pl.ds(rows0, NSL)])


_agg_kernel = pl.kernel(
    _agg_body,
    out_type=[
        jax.ShapeDtypeStruct((NC, N_PAD, D), jnp.float32),
        jax.ShapeDtypeStruct((NW, N_PAD), jnp.float32),
    ],
    mesh=_MESH,
    scratch_types=[
        pltpu.VMEM((N_PAD,), jnp.float32),
        pltpu.VMEM((N_PAD,), jnp.float32),
        pltpu.VMEM((NSL,), jnp.float32),
        pltpu.VMEM((NSL,), jnp.float32),
        pltpu.VMEM((4 * CHC,), jnp.int32),
        pltpu.VMEM((4 * CHC,), jnp.int32),
        pltpu.VMEM((CHC,), jnp.int32),
        pltpu.VMEM((CHC,), jnp.int32),
        pltpu.VMEM((CHC,), jnp.float32),
        pltpu.VMEM((CHC,), jnp.float32),
        pltpu.VMEM((CHC, D), jnp.float32),
        pltpu.VMEM((CHC, D), jnp.float32),
        pltpu.VMEM((CHC, D), jnp.float32),
        pltpu.VMEM((CHC,), jnp.int32),
        pltpu.VMEM((CHC,), jnp.int32),
        pltpu.VMEM((3 * L,), jnp.int32),
        pltpu.VMEM((2 * L,), jnp.float32),
        pltpu.VMEM_SHARED((N_PAD, D), jnp.float32),
        pltpu.VMEM_SHARED((N_PAD,), jnp.float32),
        pltpu.SemaphoreType.DMA,
        pltpu.SemaphoreType.DMA,
        pltpu.SemaphoreType.DMA,
    ],
    compiler_params=pltpu.CompilerParams(needs_layout_passes=False),
)


def _out_body(nt_ref, tp_ref, dp_ref, aw_ref, alpha_ref, out_ref):
    t = tp_ref[0] + tp_ref[1]
    dp = dp_ref[...]
    den = lax.dot_general(dp, jnp.ones((NW, 1), jnp.float32),
                          (((1,), (0,)), ((), ())),
                          preferred_element_type=jnp.float32)
    safe = den > 0
    x = jnp.where(safe, t / jnp.where(safe, den, 1.0), 0.0)
    nt = nt_ref[...]
    acc = jnp.zeros((x.shape[0], D), jnp.float32)
    alpha = jnp.zeros((x.shape[0], 1), jnp.float32)
    for tt in range(NT):
        m = (nt == tt).astype(jnp.float32)
        acc += m * jnp.dot(x, aw_ref[tt], preferred_element_type=jnp.float32)
        alpha += m * alpha_ref[tt]
    out_ref[...] = acc * alpha


def _final(nt2, t_part, den_part, aw, alpha8):
    grid = N // _BN
    return pl.pallas_call(
        _out_body,
        grid=(grid,),
        in_specs=[
            pl.BlockSpec((_BN, 1), lambda i: (i, 0)),
            pl.BlockSpec((NC, _BN, D), lambda i: (0, i, 0)),
            pl.BlockSpec((_BN, NW), lambda i: (i, 0)),
            pl.BlockSpec((NT, D, D), lambda i: (0, 0, 0)),
            pl.BlockSpec((8, 1), lambda i: (0, 0)),
        ],
        out_specs=pl.BlockSpec((_BN, D), lambda i: (i, 0)),
        out_shape=jax.ShapeDtypeStruct((N, D), jnp.float32),
    )(nt2, t_part, den_part, aw, alpha8)


def kernel(h, k_weight, q_weight, v_weight, a_weight, relation_pri,
           relation_att, relation_msg, skip, edge_index, node_type,
           edge_type):
    nt2 = node_type.reshape(N, 1)
    q_node, k_all, v_all = _proj(nt2, h, k_weight, q_weight, v_weight,
                                 relation_att, relation_msg)
    k_flat = k_all.reshape(NR * N, D)
    v_flat = v_all.reshape(NR * N, D)

    aux3 = jnp.concatenate(
        [edge_index, edge_type.reshape(1, E).astype(jnp.int32)], axis=0)
    aux3c = aux3.reshape(3, NCHUNK, CH).transpose(1, 0, 2).reshape(
        NCHUNK, 3 * CH)
    pri16 = jnp.zeros((L,), jnp.float32).at[:NR].set(
        relation_pri.reshape(NR).astype(jnp.float32))

    attn, m_part = _attn_kernel(k_flat, q_node, aux3c, pri16)
    aux4 = jnp.concatenate(
        [aux3, jax.lax.bitcast_convert_type(attn, jnp.int32).reshape(1, E)],
        axis=0)
    aux4c = aux4.reshape(4, NCHUNKC, CHC).transpose(1, 0, 2).reshape(
        NCHUNKC, 4 * CHC)
    t_part, den_part = _agg_kernel(v_flat, aux4c, m_part)
    den_t = den_part.T

    alpha8 = jnp.zeros((8, 1), jnp.float32).at[:NT, 0].set(
        jax.nn.sigmoid(skip.astype(jnp.float32)))
    return _final(nt2, t_part, den_t, a_weight, alpha8)


# speedup vs baseline: 12.9810x; 1.0630x over previous
import functools
import math

import jax
import jax.numpy as jnp
from jax import lax
from jax.experimental import pallas as pl
from jax.experimental.pallas import tpu as pltpu
from jax.experimental.pallas import tpu_sc as plsc

N = 10000
E = 160000
D = 128
NT = 4
NR = 4

NC = 2
NS = 16
NW = NC * NS
L = 16

CH = 128
NCHUNK = E // CH
CHC = 64
NCHUNKC = E // CHC
N_PAD = 10240
NSL = N_PAD // NS

_INV_SQRT_DK = 1.0 / math.sqrt(float(D))
_NEG_BIG = -3.0e38


_BN = 1000


def _proj_body(nt_ref, h_ref, kw_ref, qw_ref, vw_ref, att_ref, msg_ref,
               q_out, k_out, v_out):
    h = h_ref[...]
    nt = nt_ref[...]
    kn = jnp.zeros_like(h)
    qn = jnp.zeros_like(h)
    vn = jnp.zeros_like(h)
    for t in range(NT):
        m = (nt == t).astype(jnp.float32)
        kn += m * jnp.dot(h, kw_ref[t], preferred_element_type=jnp.float32)
        qn += m * jnp.dot(h, qw_ref[t], preferred_element_type=jnp.float32)
        vn += m * jnp.dot(h, vw_ref[t], preferred_element_type=jnp.float32)
    q_out[...] = qn
    for r in range(NR):
        k_out[r] = jnp.dot(kn, att_ref[r], preferred_element_type=jnp.float32)
        v_out[r] = jnp.dot(vn, msg_ref[r], preferred_element_type=jnp.float32)


def _proj(nt2, h, kw, qw, vw, att, msg):
    grid = N // _BN
    return pl.pallas_call(
        _proj_body,
        grid=(grid,),
        in_specs=[
            pl.BlockSpec((_BN, 1), lambda i: (i, 0)),
            pl.BlockSpec((_BN, D), lambda i: (i, 0)),
            pl.BlockSpec((NT, D, D), lambda i: (0, 0, 0)),
            pl.BlockSpec((NT, D, D), lambda i: (0, 0, 0)),
            pl.BlockSpec((NT, D, D), lambda i: (0, 0, 0)),
            pl.BlockSpec((NR, D, D), lambda i: (0, 0, 0)),
            pl.BlockSpec((NR, D, D), lambda i: (0, 0, 0)),
        ],
        out_specs=[
            pl.BlockSpec((_BN, D), lambda i: (i, 0)),
            pl.BlockSpec((NR, _BN, D), lambda i: (0, i, 0)),
            pl.BlockSpec((NR, _BN, D), lambda i: (0, i, 0)),
        ],
        out_shape=[
            jax.ShapeDtypeStruct((N, D), jnp.float32),
            jax.ShapeDtypeStruct((NR, N, D), jnp.float32),
            jax.ShapeDtypeStruct((NR, N, D), jnp.float32),
        ],
    )(nt2, h, kw, qw, vw, att, msg)


_MESH = plsc.VectorSubcoreMesh(core_axis_name="c", subcore_axis_name="s")


def _attn_body(kflat_hbm, qnode_hbm, aux_hbm, pri_hbm,
               attn_hbm, mpart_hbm,
               m_loc, kidx0, kidx1, auxall,
               k_rows0, k_rows1, q_rows0, q_rows1,
               attn_v0, attn_v1, pri_v, pbuf, kb, vb,
               semk0, semk1, semq0, semq1, semaux):
    c = lax.axis_index("c")
    s = lax.axis_index("s")
    w = s * NC + c
    lane = lax.iota(jnp.int32, L)
    tcol = lane * L
    zero = jnp.zeros((L,), jnp.float32)

    kidxs = (kidx0, kidx1)
    k_rowss = (k_rows0, k_rows1)
    q_rowss = (q_rows0, q_rows1)
    attn_vs = (attn_v0, attn_v1)
    semks = (semk0, semk1)
    semqs = (semq0, semq1)

    def init_body(i, _):
        m_loc[pl.ds(i * L, L)] = jnp.full((L,), _NEG_BIG, jnp.float32)
        return 0
    lax.fori_loop(0, N_PAD // L, init_body, 0)

    pltpu.sync_copy(pri_hbm, pri_v)
    kb[pl.ds(0, L)] = jnp.full((L,), -1, jnp.int32)
    kb[pl.ds(2 * L, L)] = jnp.full((L,), -2, jnp.int32)
    vb[pl.ds(0, L)] = jnp.full((L,), _NEG_BIG, jnp.float32)

    NPW_A = (NCHUNK + NW - 1) // NW
    NH_A = NPW_A // 2

    def pre_body(j, _):
        cid = lax.rem(w + j * NW, NCHUNK)
        pltpu.async_copy(aux_hbm.at[cid], auxall.at[j], semaux)
        return 0
    lax.fori_loop(0, NPW_A, pre_body, 0)

    def drain_body(j, _):
        pltpu.make_async_copy(aux_hbm.at[0], auxall.at[j], semaux).wait()
        return 0
    lax.fori_loop(0, NPW_A, drain_body, 0)

    def fire(pi, j):
        def idx_body(jj, _):
            sl = pl.ds(jj * L, L)
            kidxs[pi][sl] = (auxall[j, pl.ds(jj * L, L)]
                             + auxall[j, pl.ds(2 * CH + jj * L, L)] * N)
            return 0
        lax.fori_loop(0, CH // L, idx_body, 0)
        pltpu.async_copy(kflat_hbm.at[kidxs[pi]], k_rowss[pi], semks[pi])
        pltpu.async_copy(qnode_hbm.at[auxall.at[j, pl.ds(CH, CH)]],
                         q_rowss[pi], semqs[pi])

    def compute(pi, j):
        cid = lax.rem(w + j * NW, NCHUNK)
        base = cid * CH
        pltpu.make_async_copy(kflat_hbm.at[kidxs[pi]], k_rowss[pi],
                              semks[pi]).wait()
        pltpu.make_async_copy(qnode_hbm.at[auxall.at[j, pl.ds(CH, CH)]],
                              q_rowss[pi], semqs[pi]).wait()
        k_rows = k_rowss[pi]
        q_rows = q_rowss[pi]
        attn_v = attn_vs[pi]

        def group_body(g, _):
            eb = g * L
            sl_g = pl.ds(eb, L)
            for el in range(L):
                e = eb + el
                p = zero
                for jj in range(D // L):
                    kk = k_rows[e, pl.ds(jj * L, L)]
                    qq = q_rows[e, pl.ds(jj * L, L)]
                    p = p + kk * qq
                pbuf[pl.ds(el * L, L)] = p
            acc = zero
            for l in range(L):
                acc = acc + plsc.load_gather(pbuf, [tcol + l])
            et16 = auxall[j, pl.ds(2 * CH + eb, L)]
            pri16 = plsc.load_gather(pri_v, [et16])
            attn16 = acc * (pri16 * _INV_SQRT_DK)
            attn_v[sl_g] = attn16
            dst16 = auxall[j, pl.ds(CH + eb, L)]
            sk = plsc.sort_key_val(dst16, attn16)
            sv = sk[1]
            sk = sk[0]
            kb[pl.ds(L, L)] = sk
            vb[pl.ds(L, L)] = sv
            for shift in (1, 2, 4, 8):
                ks = kb[pl.ds(L - shift, L)]
                vs = vb[pl.ds(L - shift, L)]
                sv = jnp.maximum(sv, jnp.where(ks == sk, vs, _NEG_BIG))
                vb[pl.ds(L, L)] = sv
            nxt = kb[pl.ds(L + 1, L)]
            is_last = sk != nxt
            cur = plsc.load_gather(m_loc, [sk])
            plsc.store_scatter(m_loc, [sk], jnp.maximum(cur, sv),
                               mask=is_last)
            return 0
        lax.fori_loop(0, CH // L, group_body, 0)

        pltpu.sync_copy(attn_v, attn_hbm.at[pl.ds(base, CH)])

    fire(0, 0)

    def pair_body(i, _):
        fire(1, 2 * i + 1)
        compute(0, 2 * i)

        @pl.when(i < NH_A - 1)
        def _():
            fire(0, 2 * i + 2)
        compute(1, 2 * i + 1)
        return 0
    lax.fori_loop(0, NH_A, pair_body, 0)

    pltpu.sync_copy(m_loc, mpart_hbm.at[w])


_attn_kernel = pl.kernel(
    _attn_body,
    out_type=[
        jax.ShapeDtypeStruct((E,), jnp.float32),
        jax.ShapeDtypeStruct((NW, N_PAD), jnp.float32),
    ],
    mesh=_MESH,
    scratch_types=[
        pltpu.VMEM((N_PAD,), jnp.float32),
        pltpu.VMEM((CH,), jnp.int32),
        pltpu.VMEM((CH,), jnp.int32),
        pltpu.VMEM((40, 3 * CH), jnp.int32),
        pltpu.VMEM((CH, D), jnp.float32),
        pltpu.VMEM((CH, D), jnp.float32),
        pltpu.VMEM((CH, D), jnp.float32),
        pltpu.VMEM((CH, D), jnp.float32),
        pltpu.VMEM((CH,), jnp.float32),
        pltpu.VMEM((CH,), jnp.float32),
        pltpu.VMEM((L,), jnp.float32),
        pltpu.VMEM((L * L,), jnp.float32),
        pltpu.VMEM((3 * L,), jnp.int32),
        pltpu.VMEM((2 * L,), jnp.float32),
        pltpu.SemaphoreType.DMA,
        pltpu.SemaphoreType.DMA,
        pltpu.SemaphoreType.DMA,
        pltpu.SemaphoreType.DMA,
        pltpu.SemaphoreType.DMA,
    ],
    compiler_params=pltpu.CompilerParams(needs_layout_passes=False),
)


def _agg_body(vflat_hbm, aux_hbm, mpart_hbm,
              out_hbm, den_hbm,
              m_loc, den_loc, tmp_v, red_v,
              aux_v0, aux_v1, vidx0, vidx1,
              ex_v0, ex_v1, v_rows0, v_rows1, w2,
              dsc0, dsc1, kb, vb, t_shared, m_sh,
              semv0, semv1, semsc):
    c = lax.axis_index("c")
    s = lax.axis_index("s")
    w = s * NC + c

    aux_vs = (aux_v0, aux_v1)
    vidxs = (vidx0, vidx1)
    ex_vs = (ex_v0, ex_v1)
    v_rowss = (v_rows0, v_rows1)
    dscs = (dsc0, dsc1)
    semvs = (semv0, semv1)

    def initr_body(j, _):
        red_v[pl.ds(j * L, L)] = jnp.full((L,), _NEG_BIG, jnp.float32)
        return 0
    lax.fori_loop(0, NSL // L, initr_body, 0)

    def rrow_body(r, _):
        pltpu.sync_copy(mpart_hbm.at[r, pl.ds(s * NSL, NSL)], tmp_v)

        def rb(j, _):
            sl = pl.ds(j * L, L)
            red_v[sl] = jnp.maximum(red_v[sl], tmp_v[sl])
            return 0
        lax.fori_loop(0, NSL // L, rb, 0)
        return 0
    lax.fori_loop(0, NW, rrow_body, 0)
    pltpu.sync_copy(red_v, m_sh.at[pl.ds(s * NSL, NSL)])

    zero = jnp.zeros((L,), jnp.float32)

    def zw_body(r, _):
        for j in range(D // L):
            w2[r, pl.ds(j * L, L)] = zero
        return 0
    lax.fori_loop(0, CHC, zw_body, 0)

    def zd_body(i, _):
        den_loc[pl.ds(i * L, L)] = zero
        return 0
    lax.fori_loop(0, N_PAD // L, zd_body, 0)

    kb[pl.ds(0, L)] = jnp.full((L,), -1, jnp.int32)
    kb[pl.ds(2 * L, L)] = jnp.full((L,), -2, jnp.int32)
    vb[pl.ds(0, L)] = zero

    rows0 = s * NSL
    for blk in range(NSL // CHC):
        pltpu.sync_copy(w2, t_shared.at[pl.ds(rows0 + blk * CHC, CHC)])

    plsc.subcore_barrier()
    pltpu.sync_copy(m_sh, m_loc)

    nchunks = jnp.where(w < NCHUNKC - (NCHUNKC // NW) * NW,
                        NCHUNKC // NW + 1, NCHUNKC // NW)
    NH_C = (NCHUNKC // NW + 2) // 2

    def fire(pi, j):
        cid = w + j * NW
        pltpu.sync_copy(aux_hbm.at[cid], aux_vs[pi])

        def idx_body(jj, _):
            sl = pl.ds(jj * L, L)
            vidxs[pi][sl] = (aux_vs[pi][pl.ds(jj * L, L)]
                             + aux_vs[pi][pl.ds(2 * CHC + jj * L, L)] * N)
            return 0
        lax.fori_loop(0, CHC // L, idx_body, 0)
        pltpu.async_copy(vflat_hbm.at[vidxs[pi]], v_rowss[pi], semvs[pi])

    def compute(pi, j):
        ex_v = ex_vs[pi]
        aux_v = aux_vs[pi]
        v_rows = v_rowss[pi]

        def ex_body(jj, _):
            md = plsc.load_gather(m_loc, [aux_v[pl.ds(CHC + jj * L, L)]])
            at16 = plsc.bitcast(aux_v[pl.ds(3 * CHC + jj * L, L)],
                                jnp.float32)
            ex_v[pl.ds(jj * L, L)] = jnp.exp(at16 - md)
            return 0
        lax.fori_loop(0, CHC // L, ex_body, 0)

        @pl.when(j >= 1)
        def _():
            pltpu.make_async_copy(w2, t_shared.at[dscs[1 - pi]],
                                  semsc).wait()
        pltpu.make_async_copy(vflat_hbm.at[vidxs[pi]], v_rowss[pi],
                              semvs[pi]).wait()

        def group_body(g, _):
            eb = g * L
            sl_g = pl.ds(eb, L)
            x16 = ex_v[sl_g]
            for el in range(L):
                x = x16[el]
                e = eb + el
                for jj in range(D // L):
                    w2[e, pl.ds(jj * L, L)] = v_rows[e, pl.ds(jj * L, L)] * x
            dst16 = aux_v[pl.ds(CHC + eb, L)]
            sk_sv = plsc.sort_key_val(dst16, x16)
            sk = sk_sv[0]
            sv = sk_sv[1]
            kb[pl.ds(L, L)] = sk
            vb[pl.ds(L, L)] = sv
            for shift in (1, 2, 4, 8):
                ks = kb[pl.ds(L - shift, L)]
                vs = vb[pl.ds(L - shift, L)]
                sv = sv + jnp.where(ks == sk, vs, 0.0)
                vb[pl.ds(L, L)] = sv
            nxt = kb[pl.ds(L + 1, L)]
            is_last = sk != nxt
            plsc.addupdate_scatter(den_loc, [sk], sv, mask=is_last)
            return 0
        lax.fori_loop(0, CHC // L, group_body, 0)

        def cp_body(jj, _):
            sl = pl.ds(jj * L, L)
            dscs[pi][sl] = aux_v[pl.ds(CHC + jj * L, L)]
            return 0
        lax.fori_loop(0, CHC // L, cp_body, 0)
        pltpu.async_copy(w2, t_shared.at[dscs[pi]], semsc, add=True)

    fire(0, 0)

    def pair_body(i, _):
        a = 2 * i
        b = 2 * i + 1

        @pl.when(b < nchunks)
        def _():
            fire(1, b)

        compute(0, a)

        @pl.when(a + 2 < nchunks)
        def _():
            fire(0, a + 2)

        @pl.when(b < nchunks)
        def _():
            compute(1, b)
        return 0
    lax.fori_loop(0, (nchunks + 1) // 2, pair_body, 0)

    p_last = lax.rem(nchunks - 1, 2)

    @pl.when(p_last == 0)
    def _():
        pltpu.make_async_copy(w2, t_shared.at[dsc0], semsc).wait()

    @pl.when(p_last == 1)
    def _():
        pltpu.make_async_copy(w2, t_shared.at[dsc1], semsc).wait()

    pltpu.sync_copy(den_loc, den_hbm.at[w])
    plsc.subcore_barrier()
    pltpu.sync_copy(t_shared.at[pl.ds(rows0, NSL)],
                    out_hbm.at[c, pl.ds(rows0, NSL)])


_agg_kernel = pl.kernel(
    _agg_body,
    out_type=[
        jax.ShapeDtypeStruct((NC, N_PAD, D), jnp.float32),
        jax.ShapeDtypeStruct((NW, N_PAD), jnp.float32),
    ],
    mesh=_MESH,
    scratch_types=[
        pltpu.VMEM((N_PAD,), jnp.float32),
        pltpu.VMEM((N_PAD,), jnp.float32),
        pltpu.VMEM((NSL,), jnp.float32),
        pltpu.VMEM((NSL,), jnp.float32),
        pltpu.VMEM((4 * CHC,), jnp.int32),
        pltpu.VMEM((4 * CHC,), jnp.int32),
        pltpu.VMEM((CHC,), jnp.int32),
        pltpu.VMEM((CHC,), jnp.int32),
        pltpu.VMEM((CHC,), jnp.float32),
        pltpu.VMEM((CHC,), jnp.float32),
        pltpu.VMEM((CHC, D), jnp.float32),
        pltpu.VMEM((CHC, D), jnp.float32),
        pltpu.VMEM((CHC, D), jnp.float32),
        pltpu.VMEM((CHC,), jnp.int32),
        pltpu.VMEM((CHC,), jnp.int32),
        pltpu.VMEM((3 * L,), jnp.int32),
        pltpu.VMEM((2 * L,), jnp.float32),
        pltpu.VMEM_SHARED((N_PAD, D), jnp.float32),
        pltpu.VMEM_SHARED((N_PAD,), jnp.float32),
        pltpu.SemaphoreType.DMA,
        pltpu.SemaphoreType.DMA,
        pltpu.SemaphoreType.DMA,
    ],
    compiler_params=pltpu.CompilerParams(needs_layout_passes=False),
)


def _out_body(nt_ref, tp_ref, dp_ref, aw_ref, alpha_ref, out_ref):
    t = tp_ref[0] + tp_ref[1]
    dp = dp_ref[...]
    den = lax.dot_general(dp, jnp.ones((NW, 1), jnp.float32),
                          (((1,), (0,)), ((), ())),
                          preferred_element_type=jnp.float32)
    safe = den > 0
    x = jnp.where(safe, t / jnp.where(safe, den, 1.0), 0.0)
    nt = nt_ref[...]
    acc = jnp.zeros((x.shape[0], D), jnp.float32)
    alpha = jnp.zeros((x.shape[0], 1), jnp.float32)
    for tt in range(NT):
        m = (nt == tt).astype(jnp.float32)
        acc += m * jnp.dot(x, aw_ref[tt], preferred_element_type=jnp.float32)
        alpha += m * alpha_ref[tt]
    out_ref[...] = acc * alpha


def _final(nt2, t_part, den_part, aw, alpha8):
    grid = N // _BN
    return pl.pallas_call(
        _out_body,
        grid=(grid,),
        in_specs=[
            pl.BlockSpec((_BN, 1), lambda i: (i, 0)),
            pl.BlockSpec((NC, _BN, D), lambda i: (0, i, 0)),
            pl.BlockSpec((_BN, NW), lambda i: (i, 0)),
            pl.BlockSpec((NT, D, D), lambda i: (0, 0, 0)),
            pl.BlockSpec((8, 1), lambda i: (0, 0)),
        ],
        out_specs=pl.BlockSpec((_BN, D), lambda i: (i, 0)),
        out_shape=jax.ShapeDtypeStruct((N, D), jnp.float32),
    )(nt2, t_part, den_part, aw, alpha8)


def kernel(h, k_weight, q_weight, v_weight, a_weight, relation_pri,
           relation_att, relation_msg, skip, edge_index, node_type,
           edge_type):
    nt2 = node_type.reshape(N, 1)
    q_node, k_all, v_all = _proj(nt2, h, k_weight, q_weight, v_weight,
                                 relation_att, relation_msg)
    k_flat = k_all.reshape(NR * N, D)
    v_flat = v_all.reshape(NR * N, D)

    aux3 = jnp.concatenate(
        [edge_index, edge_type.reshape(1, E).astype(jnp.int32)], axis=0)
    aux3c = aux3.reshape(3, NCHUNK, CH).transpose(1, 0, 2).reshape(
        NCHUNK, 3 * CH)
    pri16 = jnp.zeros((L,), jnp.float32).at[:NR].set(
        relation_pri.reshape(NR).astype(jnp.float32))

    attn, m_part = _attn_kernel(k_flat, q_node, aux3c, pri16)
    aux4 = jnp.concatenate(
        [aux3, jax.lax.bitcast_convert_type(attn, jnp.int32).reshape(1, E)],
        axis=0)
    aux4c = aux4.reshape(4, NCHUNKC, CHC).transpose(1, 0, 2).reshape(
        NCHUNKC, 4 * CHC)
    t_part, den_part = _agg_kernel(v_flat, aux4c, m_part)
    den_t = den_part.T

    alpha8 = jnp.zeros((8, 1), jnp.float32).at[:NT, 0].set(
        jax.nn.sigmoid(skip.astype(jnp.float32)))
    return _final(nt2, t_part, den_t, a_weight, alpha8)
